# Initial kernel scaffold; baseline (speedup 1.0000x reference)
import jax
import jax.numpy as jnp
from jax.experimental import pallas as pl


def kernel(inputs, Awij, Awij2, first_a, first_t):
    raise NotImplementedError("write your pallas kernel here")


# speedup vs baseline: 2.4086x; 2.4086x over previous
import jax
import jax.numpy as jnp
from jax.experimental import pallas as pl
from jax.experimental.pallas import tpu as pltpu

NW, NT, A, T, N_STEPS = 4096, 8192, 8, 2, 5
BLK = 256
NB = NW // BLK


def _hi_lo(x):
    hi = x.astype(jnp.bfloat16)
    lo = (x - hi.astype(jnp.float32)).astype(jnp.bfloat16)
    return jnp.concatenate([hi, lo], axis=1)


def _body(inp_ref, a2_ref, d_ref, w0_ref, a0_ref, t0_ref, a_out, t_out,
          a_s, t_s, v_s, cs_s):
    s = pl.program_id(0)
    i = pl.program_id(1)

    @pl.when(jnp.logical_and(s == 0, i == 0))
    def _init():
        a_s[...] = a0_ref[...]
        t_s[...] = t0_ref[...]

    @pl.when(i == 0)
    def _zero_acc():
        v_s[...] = jnp.zeros_like(v_s)
        cs_s[...] = jnp.zeros_like(cs_s)

    m = inp_ref[...].astype(jnp.bfloat16)
    t = t_s[...]
    a2 = a2_ref[...]
    d = d_ref[...]

    s0 = jnp.sum(t[:, 0:1])
    uu = jax.lax.dot_general(m, _hi_lo(t), (((1,), (0,)), ((), ())),
                             preferred_element_type=jnp.float32)
    u = uu[:, :2] + uu[:, 2:]
    a_blk = a_s[pl.ds(i * BLK, BLK), :]
    a_blk = a_blk + (s0 - u[:, 0:1]) * a2[0:1, :] + u[:, 1:2] * a2[1:2, :]
    a_s[pl.ds(i * BLK, BLK), :] = a_blk

    g0 = jnp.sum(a_blk * d[:, 0][None, :], axis=1, keepdims=True)
    g1 = jnp.sum(a_blk * d[:, 1][None, :], axis=1, keepdims=True)
    g = jnp.concatenate([g0, g1], axis=1)
    cs_s[...] += jnp.sum(a_blk, axis=0, keepdims=True)
    vv = jax.lax.dot_general(m, _hi_lo(g), (((0,), (0,)), ((), ())),
                             preferred_element_type=jnp.float32)
    v_s[...] += vv[:, :2] + vv[:, 2:]

    @pl.when(i == NB - 1)
    def _finish_step():
        w0 = w0_ref[...]
        cs = cs_s[...]
        c0 = jnp.sum(cs * w0[:, 0][None, :], axis=1, keepdims=True)
        c1 = jnp.sum(cs * w0[:, 1][None, :], axis=1, keepdims=True)
        t_s[...] = t_s[...] + jnp.concatenate([c0, c1], axis=1) + v_s[...]

    @pl.when(jnp.logical_and(s == N_STEPS - 1, i == NB - 1))
    def _finish():
        a_out[...] = a_s[...]
        t_out[...] = t_s[...]


@jax.jit
def kernel(inputs, Awij, Awij2, first_a, first_t):
    a2 = Awij2[:, 0, :]
    d = Awij[1] - Awij[0]
    w0 = Awij[0]
    full = lambda shape: pl.BlockSpec(shape, lambda s, i: (0, 0))
    a_out, t_out = pl.pallas_call(
        _body,
        grid=(N_STEPS, NB),
        in_specs=[
            pl.BlockSpec((BLK, NT), lambda s, i: (i, 0)),
            full((T, A)), full((A, T)), full((A, T)),
            full((NW, A)), full((NT, T)),
        ],
        out_specs=[full((NW, A)), full((NT, T))],
        out_shape=[
            jax.ShapeDtypeStruct((NW, A), jnp.float32),
            jax.ShapeDtypeStruct((NT, T), jnp.float32),
        ],
        scratch_shapes=[
            pltpu.VMEM((NW, A), jnp.float32),
            pltpu.VMEM((NT, T), jnp.float32),
            pltpu.VMEM((NT, T), jnp.float32),
            pltpu.VMEM((1, A), jnp.float32),
        ],
        compiler_params=pltpu.CompilerParams(
            dimension_semantics=("arbitrary", "arbitrary"),
            vmem_limit_bytes=60 * 1024 * 1024,
        ),
    )(inputs, a2, d, w0, first_a, first_t)
    return a_out, t_out


# speedup vs baseline: 2.6428x; 1.0972x over previous
import jax
import jax.numpy as jnp
from jax.experimental import pallas as pl
from jax.experimental.pallas import tpu as pltpu

NW, NT, A, T, N_STEPS = 4096, 8192, 8, 2, 5
BLK1 = 256
NB1 = NW // BLK1
BLK2 = 512
NB2 = NW // BLK2


def _hi_lo_cols(x):
    hi = x.astype(jnp.bfloat16)
    lo = (x - hi.astype(jnp.float32)).astype(jnp.bfloat16)
    return jnp.concatenate([hi, lo], axis=1)


def _step_block(m, i, blk, rhs_ref, a_s, v_s, cs_s, a2_ref, d_ref, t_row0):
    s0 = jnp.sum(t_row0)
    uu = jax.lax.dot_general(m, rhs_ref[...], (((1,), (0,)), ((), ())),
                             preferred_element_type=jnp.float32)
    u = uu[:, :2] + uu[:, 2:]
    a2 = a2_ref[...]
    a_blk = a_s[pl.ds(i * blk, blk), :]
    a_blk = a_blk + (s0 - u[:, 0:1]) * a2[0:1, :] + u[:, 1:2] * a2[1:2, :]
    a_s[pl.ds(i * blk, blk), :] = a_blk
    d = d_ref[...]
    g0 = jnp.sum(a_blk * d[:, 0][None, :], axis=1, keepdims=True)
    g1 = jnp.sum(a_blk * d[:, 1][None, :], axis=1, keepdims=True)
    ghl = _hi_lo_cols(jnp.concatenate([g0, g1], axis=1))
    vv = jax.lax.dot_general(ghl, m, (((0,), (0,)), ((), ())),
                             preferred_element_type=jnp.float32)
    v_s[...] += vv
    cs_s[...] += jnp.sum(a_blk, axis=0, keepdims=True)


def _finish_t(w0_ref, cs_s, v_s, t_s):
    w0 = w0_ref[...]
    cs = cs_s[...]
    c0 = jnp.sum(cs * w0[:, 0][None, :])
    c1 = jnp.sum(cs * w0[:, 1][None, :])
    v = v_s[:2, :] + v_s[2:, :]
    c = jnp.concatenate([jnp.full((1, NT), c0, jnp.float32),
                         jnp.full((1, NT), c1, jnp.float32)], axis=0)
    t_s[...] = t_s[...] + c + v


def _rhs_from_t(t_s, rhs_ref):
    t_wide = t_s[...]
    hi = t_wide.astype(jnp.bfloat16)
    lo = (t_wide - hi.astype(jnp.float32)).astype(jnp.bfloat16)
    rhs_ref[...] = jnp.concatenate([hi, lo], axis=0).T


def _body1(inp_ref, a2_ref, d_ref, w0_ref, a0_ref, t0_ref,
           m16_out, a_out, t_out, v_s, cs_s, t_s, rhs_s):
    i = pl.program_id(0)

    @pl.when(i == 0)
    def _init():
        t0 = t0_ref[...]
        rhs_s[...] = _hi_lo_cols(t0)
        t_s[...] = t0.T
        a_out[...] = a0_ref[...]
        v_s[...] = jnp.zeros_like(v_s)
        cs_s[...] = jnp.zeros_like(cs_s)

    m = inp_ref[...].astype(jnp.bfloat16)
    m16_out[...] = m
    _step_block(m, i, BLK1, rhs_s, a_out, v_s, cs_s, a2_ref, d_ref,
                t_s[0:1, :])

    @pl.when(i == NB1 - 1)
    def _finish():
        _finish_t(w0_ref, cs_s, v_s, t_s)
        t_out[...] = t_s[...]


def _body2(m16_ref, a2_ref, d_ref, w0_ref, a1_ref, t1_ref,
           a_out, t_out, a_s, v_s, cs_s, t_s, rhs_s):
    s = pl.program_id(0)
    i = pl.program_id(1)

    @pl.when(jnp.logical_and(s == 0, i == 0))
    def _init():
        a_s[...] = a1_ref[...]
        t_s[...] = t1_ref[...]
        _rhs_from_t(t_s, rhs_s)

    @pl.when(i == 0)
    def _zero():
        v_s[...] = jnp.zeros_like(v_s)
        cs_s[...] = jnp.zeros_like(cs_s)

    _step_block(m16_ref[...], i, BLK2, rhs_s, a_s, v_s, cs_s, a2_ref, d_ref,
                t_s[0:1, :])

    @pl.when(i == NB2 - 1)
    def _finish_step():
        _finish_t(w0_ref, cs_s, v_s, t_s)
        _rhs_from_t(t_s, rhs_s)

    @pl.when(jnp.logical_and(s == N_STEPS - 2, i == NB2 - 1))
    def _finish():
        a_out[...] = a_s[...]
        t_out[...] = t_s[...].T


@jax.jit
def kernel(inputs, Awij, Awij2, first_a, first_t):
    a2 = Awij2[:, 0, :]
    d = Awij[1] - Awij[0]
    w0 = Awij[0]
    f32 = jnp.float32

    full1 = lambda shape: pl.BlockSpec(shape, lambda i: (0, 0))
    m16, a1, t1 = pl.pallas_call(
        _body1,
        grid=(NB1,),
        in_specs=[
            pl.BlockSpec((BLK1, NT), lambda i: (i, 0)),
            full1((T, A)), full1((A, T)), full1((A, T)),
            full1((NW, A)), full1((NT, T)),
        ],
        out_specs=[pl.BlockSpec((BLK1, NT), lambda i: (i, 0)),
                   full1((NW, A)), full1((T, NT))],
        out_shape=[jax.ShapeDtypeStruct((NW, NT), jnp.bfloat16),
                   jax.ShapeDtypeStruct((NW, A), f32),
                   jax.ShapeDtypeStruct((T, NT), f32)],
        scratch_shapes=[
            pltpu.VMEM((4, NT), f32),
            pltpu.VMEM((1, A), f32),
            pltpu.VMEM((T, NT), f32),
            pltpu.VMEM((NT, 2 * T), jnp.bfloat16),
        ],
        compiler_params=pltpu.CompilerParams(
            dimension_semantics=("arbitrary",),
            vmem_limit_bytes=60 * 1024 * 1024,
        ),
    )(inputs, a2, d, w0, first_a, first_t)

    full2 = lambda shape: pl.BlockSpec(shape, lambda s, i: (0, 0))
    a_out, t_out = pl.pallas_call(
        _body2,
        grid=(N_STEPS - 1, NB2),
        in_specs=[
            pl.BlockSpec((BLK2, NT), lambda s, i: (i, 0)),
            full2((T, A)), full2((A, T)), full2((A, T)),
            full2((NW, A)), full2((T, NT)),
        ],
        out_specs=[full2((NW, A)), full2((NT, T))],
        out_shape=[jax.ShapeDtypeStruct((NW, A), f32),
                   jax.ShapeDtypeStruct((NT, T), f32)],
        scratch_shapes=[
            pltpu.VMEM((NW, A), f32),
            pltpu.VMEM((4, NT), f32),
            pltpu.VMEM((1, A), f32),
            pltpu.VMEM((T, NT), f32),
            pltpu.VMEM((NT, 2 * T), jnp.bfloat16),
        ],
        compiler_params=pltpu.CompilerParams(
            dimension_semantics=("arbitrary", "arbitrary"),
            vmem_limit_bytes=60 * 1024 * 1024,
        ),
    )(m16, a2, d, w0, a1, t1)
    return a_out, t_out


# speedup vs baseline: 3.7467x; 1.4177x over previous
import jax
import jax.numpy as jnp
from jax.experimental import pallas as pl
from jax.experimental.pallas import tpu as pltpu

NW, NT, A, T, N_STEPS = 4096, 8192, 8, 2, 5
BLK1 = 512
NB1 = NW // BLK1
BLK2 = 1024
NB2 = NW // BLK2


def _hi_lo_cols(x):
    hi = x.astype(jnp.bfloat16)
    lo = (x - hi.astype(jnp.float32)).astype(jnp.bfloat16)
    return jnp.concatenate([hi, lo], axis=1)


def _u_chain(m, s0, a_blk, rhs, a2, d):
    uu = jax.lax.dot_general(m, rhs, (((1,), (0,)), ((), ())),
                             preferred_element_type=jnp.float32)
    u = uu[:, :2] + uu[:, 2:]
    a_blk = a_blk + (s0 - u[:, 0:1]) * a2[0:1, :] + u[:, 1:2] * a2[1:2, :]
    g0 = jnp.sum(a_blk * d[:, 0][None, :], axis=1, keepdims=True)
    g1 = jnp.sum(a_blk * d[:, 1][None, :], axis=1, keepdims=True)
    return a_blk, jnp.concatenate([g0, g1], axis=1)


def _init_state(a_src, t_src, a_dst, t_s, rhs_s, s0_s):
    a_dst[...] = a_src[...]
    t0 = t_src[...]
    t_s[...] = t0
    hi = t0.astype(jnp.bfloat16)
    lo = (t0 - hi.astype(jnp.float32)).astype(jnp.bfloat16)
    rhs_s[...] = jnp.concatenate([hi, lo], axis=0).T
    s0_s[0] = jnp.sum(t0[0:1, :])


def _quarters(m_vals, base, q, a_state, rhs, a2, d, s0, v_s, cs_s):
    cs = jnp.zeros((1, A), jnp.float32)
    vv = None
    for k, m in enumerate(m_vals):
        aq = a_state[pl.ds(base + k * q, q), :]
        aq, gq = _u_chain(m, s0, aq, rhs, a2, d)
        vq = jax.lax.dot_general(_hi_lo_cols(gq), m,
                                 (((0,), (0,)), ((), ())),
                                 preferred_element_type=jnp.float32)
        a_state[pl.ds(base + k * q, q), :] = aq
        vv = vq if vv is None else vv + vq
        cs = cs + jnp.sum(aq, axis=0, keepdims=True)
    v_s[...] += vv
    cs_s[...] += cs


def _finish_t(w0_ref, cs_s, v_s, t_s, rhs_s, s0_s):
    w0 = w0_ref[...]
    cs = cs_s[...]
    c0 = jnp.sum(cs * w0[:, 0][None, :])
    c1 = jnp.sum(cs * w0[:, 1][None, :])
    v = v_s[:2, :] + v_s[2:, :]
    c = jnp.concatenate([jnp.full((1, NT), c0, jnp.float32),
                         jnp.full((1, NT), c1, jnp.float32)], axis=0)
    t_new = t_s[...] + c + v
    t_s[...] = t_new
    hi = t_new.astype(jnp.bfloat16)
    lo = (t_new - hi.astype(jnp.float32)).astype(jnp.bfloat16)
    rhs_s[...] = jnp.concatenate([hi, lo], axis=0).T
    s0_s[0] = jnp.sum(t_new[0:1, :])


def _body1(i0_ref, i1_ref, i2_ref, i3_ref, a2_ref, d_ref, w0_ref,
           a0_ref, t0_ref, m16_out, a_out, t_out,
           v_s, cs_s, t_s, rhs_s, s0_s):
    i = pl.program_id(0)
    q = BLK1 // 4

    @pl.when(i == 0)
    def _init():
        _init_state(a0_ref, t0_ref, a_out, t_s, rhs_s, s0_s)
        v_s[...] = jnp.zeros_like(v_s)
        cs_s[...] = jnp.zeros_like(cs_s)

    ms = []
    for k, r in enumerate((i0_ref, i1_ref, i2_ref, i3_ref)):
        m = r[...].astype(jnp.bfloat16)
        m16_out[pl.ds(k * q, q), :] = m
        ms.append(m)
    _quarters(ms, i * BLK1, q, a_out, rhs_s[...], a2_ref[...], d_ref[...],
              s0_s[0], v_s, cs_s)

    @pl.when(i == NB1 - 1)
    def _finish():
        _finish_t(w0_ref, cs_s, v_s, t_s, rhs_s, s0_s)
        t_out[...] = t_s[...]


def _body2(m0_ref, m1_ref, m2_ref, m3_ref, a2_ref, d_ref, w0_ref,
           a1_ref, t1_ref, a_out, t_out, a_s, v_s, cs_s, t_s, rhs_s, s0_s):
    s = pl.program_id(0)
    i = pl.program_id(1)
    q = BLK2 // 4

    @pl.when(jnp.logical_and(s == 0, i == 0))
    def _init():
        _init_state(a1_ref, t1_ref, a_s, t_s, rhs_s, s0_s)

    @pl.when(i == 0)
    def _zero():
        v_s[...] = jnp.zeros_like(v_s)
        cs_s[...] = jnp.zeros_like(cs_s)

    ms = [m0_ref[...], m1_ref[...], m2_ref[...], m3_ref[...]]
    _quarters(ms, i * BLK2, q, a_s, rhs_s[...], a2_ref[...], d_ref[...],
              s0_s[0], v_s, cs_s)

    @pl.when(i == NB2 - 1)
    def _finish_step():
        _finish_t(w0_ref, cs_s, v_s, t_s, rhs_s, s0_s)

    @pl.when(jnp.logical_and(s == N_STEPS - 2, i == NB2 - 1))
    def _finish():
        a_out[...] = a_s[...]
        t_out[...] = t_s[...]


@jax.jit
def kernel(inputs, Awij, Awij2, first_a, first_t):
    a2 = Awij2[:, 0, :]
    d = Awij[1] - Awij[0]
    w0 = Awij[0]
    f32 = jnp.float32
    q1 = BLK1 // 4

    full1 = lambda shape: pl.BlockSpec(shape, lambda i: (0, 0))
    m16, a1, t1 = pl.pallas_call(
        _body1,
        grid=(NB1,),
        in_specs=[
            pl.BlockSpec((q1, NT), lambda i: (4 * i, 0)),
            pl.BlockSpec((q1, NT), lambda i: (4 * i + 1, 0)),
            pl.BlockSpec((q1, NT), lambda i: (4 * i + 2, 0)),
            pl.BlockSpec((q1, NT), lambda i: (4 * i + 3, 0)),
            full1((T, A)), full1((A, T)), full1((A, T)),
            full1((NW, A)), full1((T, NT)),
        ],
        out_specs=[pl.BlockSpec((BLK1, NT), lambda i: (i, 0)),
                   full1((NW, A)), full1((T, NT))],
        out_shape=[jax.ShapeDtypeStruct((NW, NT), jnp.bfloat16),
                   jax.ShapeDtypeStruct((NW, A), f32),
                   jax.ShapeDtypeStruct((T, NT), f32)],
        scratch_shapes=[
            pltpu.VMEM((4, NT), f32),
            pltpu.VMEM((1, A), f32),
            pltpu.VMEM((T, NT), f32),
            pltpu.VMEM((NT, 2 * T), jnp.bfloat16),
            pltpu.SMEM((1,), f32),
        ],
        compiler_params=pltpu.CompilerParams(
            dimension_semantics=("arbitrary",),
            vmem_limit_bytes=60 * 1024 * 1024,
        ),
    )(inputs, inputs, inputs, inputs, a2, d, w0, first_a, first_t.T)

    q2 = BLK2 // 4
    full2 = lambda shape: pl.BlockSpec(shape, lambda s, i: (0, 0))
    a_out, t_wide = pl.pallas_call(
        _body2,
        grid=(N_STEPS - 1, NB2),
        in_specs=[
            pl.BlockSpec((q2, NT), lambda s, i: (4 * i, 0)),
            pl.BlockSpec((q2, NT), lambda s, i: (4 * i + 1, 0)),
            pl.BlockSpec((q2, NT), lambda s, i: (4 * i + 2, 0)),
            pl.BlockSpec((q2, NT), lambda s, i: (4 * i + 3, 0)),
            full2((T, A)), full2((A, T)), full2((A, T)),
            full2((NW, A)), full2((T, NT)),
        ],
        out_specs=[full2((NW, A)), full2((T, NT))],
        out_shape=[jax.ShapeDtypeStruct((NW, A), f32),
                   jax.ShapeDtypeStruct((T, NT), f32)],
        scratch_shapes=[
            pltpu.VMEM((NW, A), f32),
            pltpu.VMEM((4, NT), f32),
            pltpu.VMEM((1, A), f32),
            pltpu.VMEM((T, NT), f32),
            pltpu.VMEM((NT, 2 * T), jnp.bfloat16),
            pltpu.SMEM((1,), f32),
        ],
        compiler_params=pltpu.CompilerParams(
            dimension_semantics=("arbitrary", "arbitrary"),
            vmem_limit_bytes=60 * 1024 * 1024,
        ),
    )(m16, m16, m16, m16, a2, d, w0, a1, t1)
    return a_out, t_wide.T


# speedup vs baseline: 4.2739x; 1.1407x over previous
import jax
import jax.numpy as jnp
from jax.experimental import pallas as pl
from jax.experimental.pallas import tpu as pltpu

NW, NT, A, T, N_STEPS = 4096, 8192, 8, 2, 5
BLK1 = 512
NB1 = NW // BLK1
BLK2 = 1024
NB2 = NW // BLK2


def _hi_lo_cols(x):
    hi = x.astype(jnp.bfloat16)
    lo = (x - hi.astype(jnp.float32)).astype(jnp.bfloat16)
    return jnp.concatenate([hi, lo], axis=1)


def _u_chain(m, s0, a_blk, rhs, a2, d):
    uu = jax.lax.dot_general(m, rhs, (((1,), (0,)), ((), ())),
                             preferred_element_type=jnp.float32)
    u = uu[:, :2] + uu[:, 2:]
    a_blk = a_blk + (s0 - u[:, 0:1]) * a2[0:1, :] + u[:, 1:2] * a2[1:2, :]
    g0 = jnp.sum(a_blk * d[:, 0][None, :], axis=1, keepdims=True)
    g1 = jnp.sum(a_blk * d[:, 1][None, :], axis=1, keepdims=True)
    return a_blk, jnp.concatenate([g0, g1], axis=1)


def _init_state(a_src, t_src, a_dst, t_s, rhs_s, s0_s):
    a_dst[...] = a_src[...]
    t0 = t_src[...]
    t_s[...] = t0
    hi = t0.astype(jnp.bfloat16)
    lo = (t0 - hi.astype(jnp.float32)).astype(jnp.bfloat16)
    rhs_s[...] = jnp.concatenate([hi, lo], axis=0).T
    s0_s[0] = jnp.sum(t0[0:1, :])


def _quarters(m_vals, base, q, a_state, rhs, a2, d, s0, v_s, cs_s):
    cs = jnp.zeros((1, A), jnp.float32)
    vv = None
    for k, m in enumerate(m_vals):
        aq = a_state[pl.ds(base + k * q, q), :]
        aq, gq = _u_chain(m, s0, aq, rhs, a2, d)
        vq = jax.lax.dot_general(_hi_lo_cols(gq), m,
                                 (((0,), (0,)), ((), ())),
                                 preferred_element_type=jnp.float32)
        a_state[pl.ds(base + k * q, q), :] = aq
        vv = vq if vv is None else vv + vq
        cs = cs + jnp.sum(aq, axis=0, keepdims=True)
    v_s[...] += vv
    cs_s[...] += cs


def _finish_t(w0_ref, cs_s, v_s, t_s, rhs_s, s0_s):
    w0 = w0_ref[...]
    cs = cs_s[...]
    c0 = jnp.sum(cs * w0[:, 0][None, :])
    c1 = jnp.sum(cs * w0[:, 1][None, :])
    v = v_s[:2, :] + v_s[2:, :]
    c = jnp.concatenate([jnp.full((1, NT), c0, jnp.float32),
                         jnp.full((1, NT), c1, jnp.float32)], axis=0)
    t_new = t_s[...] + c + v
    t_s[...] = t_new
    hi = t_new.astype(jnp.bfloat16)
    lo = (t_new - hi.astype(jnp.float32)).astype(jnp.bfloat16)
    rhs_s[...] = jnp.concatenate([hi, lo], axis=0).T
    s0_s[0] = jnp.sum(t_new[0:1, :])


def _body1(i0_ref, i1_ref, i2_ref, i3_ref, a2_ref, d_ref, w0_ref,
           a0_ref, t0_ref, m16_out, a_out, t_out,
           v_s, cs_s, t_s, rhs_s, s0_s):
    i = pl.program_id(0)
    q = BLK1 // 4

    @pl.when(i == 0)
    def _init():
        _init_state(a0_ref, t0_ref, a_out, t_s, rhs_s, s0_s)
        v_s[...] = jnp.zeros_like(v_s)
        cs_s[...] = jnp.zeros_like(cs_s)

    ms = []
    for k, r in enumerate((i0_ref, i1_ref, i2_ref, i3_ref)):
        m = r[...].astype(jnp.float8_e4m3fn)
        m16_out[pl.ds(k * q, q), :] = m
        ms.append(m)
    _quarters(ms, i * BLK1, q, a_out, rhs_s[...], a2_ref[...], d_ref[...],
              s0_s[0], v_s, cs_s)

    @pl.when(i == NB1 - 1)
    def _finish():
        _finish_t(w0_ref, cs_s, v_s, t_s, rhs_s, s0_s)
        t_out[...] = t_s[...]


def _body2(m0_ref, m1_ref, m2_ref, m3_ref, a2_ref, d_ref, w0_ref,
           a1_ref, t1_ref, a_out, t_out, a_s, v_s, cs_s, t_s, rhs_s, s0_s):
    s = pl.program_id(0)
    i = pl.program_id(1)
    q = BLK2 // 4

    @pl.when(jnp.logical_and(s == 0, i == 0))
    def _init():
        _init_state(a1_ref, t1_ref, a_s, t_s, rhs_s, s0_s)

    @pl.when(i == 0)
    def _zero():
        v_s[...] = jnp.zeros_like(v_s)
        cs_s[...] = jnp.zeros_like(cs_s)

    ms = [m0_ref[...], m1_ref[...], m2_ref[...], m3_ref[...]]
    _quarters(ms, i * BLK2, q, a_s, rhs_s[...], a2_ref[...], d_ref[...],
              s0_s[0], v_s, cs_s)

    @pl.when(i == NB2 - 1)
    def _finish_step():
        _finish_t(w0_ref, cs_s, v_s, t_s, rhs_s, s0_s)

    @pl.when(jnp.logical_and(s == N_STEPS - 2, i == NB2 - 1))
    def _finish():
        a_out[...] = a_s[...]
        t_out[...] = t_s[...]


@jax.jit
def kernel(inputs, Awij, Awij2, first_a, first_t):
    a2 = Awij2[:, 0, :]
    d = Awij[1] - Awij[0]
    w0 = Awij[0]
    f32 = jnp.float32
    q1 = BLK1 // 4

    full1 = lambda shape: pl.BlockSpec(shape, lambda i: (0, 0))
    m16, a1, t1 = pl.pallas_call(
        _body1,
        grid=(NB1,),
        in_specs=[
            pl.BlockSpec((q1, NT), lambda i: (4 * i, 0)),
            pl.BlockSpec((q1, NT), lambda i: (4 * i + 1, 0)),
            pl.BlockSpec((q1, NT), lambda i: (4 * i + 2, 0)),
            pl.BlockSpec((q1, NT), lambda i: (4 * i + 3, 0)),
            full1((T, A)), full1((A, T)), full1((A, T)),
            full1((NW, A)), full1((T, NT)),
        ],
        out_specs=[pl.BlockSpec((BLK1, NT), lambda i: (i, 0)),
                   full1((NW, A)), full1((T, NT))],
        out_shape=[jax.ShapeDtypeStruct((NW, NT), jnp.float8_e4m3fn),
                   jax.ShapeDtypeStruct((NW, A), f32),
                   jax.ShapeDtypeStruct((T, NT), f32)],
        scratch_shapes=[
            pltpu.VMEM((4, NT), f32),
            pltpu.VMEM((1, A), f32),
            pltpu.VMEM((T, NT), f32),
            pltpu.VMEM((NT, 2 * T), jnp.bfloat16),
            pltpu.SMEM((1,), f32),
        ],
        compiler_params=pltpu.CompilerParams(
            dimension_semantics=("arbitrary",),
            vmem_limit_bytes=60 * 1024 * 1024,
        ),
    )(inputs, inputs, inputs, inputs, a2, d, w0, first_a, first_t.T)

    q2 = BLK2 // 4
    full2 = lambda shape: pl.BlockSpec(shape, lambda s, i: (0, 0))
    a_out, t_wide = pl.pallas_call(
        _body2,
        grid=(N_STEPS - 1, NB2),
        in_specs=[
            pl.BlockSpec((q2, NT), lambda s, i: (4 * i, 0)),
            pl.BlockSpec((q2, NT), lambda s, i: (4 * i + 1, 0)),
            pl.BlockSpec((q2, NT), lambda s, i: (4 * i + 2, 0)),
            pl.BlockSpec((q2, NT), lambda s, i: (4 * i + 3, 0)),
            full2((T, A)), full2((A, T)), full2((A, T)),
            full2((NW, A)), full2((T, NT)),
        ],
        out_specs=[full2((NW, A)), full2((T, NT))],
        out_shape=[jax.ShapeDtypeStruct((NW, A), f32),
                   jax.ShapeDtypeStruct((T, NT), f32)],
        scratch_shapes=[
            pltpu.VMEM((NW, A), f32),
            pltpu.VMEM((4, NT), f32),
            pltpu.VMEM((1, A), f32),
            pltpu.VMEM((T, NT), f32),
            pltpu.VMEM((NT, 2 * T), jnp.bfloat16),
            pltpu.SMEM((1,), f32),
        ],
        compiler_params=pltpu.CompilerParams(
            dimension_semantics=("arbitrary", "arbitrary"),
            vmem_limit_bytes=60 * 1024 * 1024,
        ),
    )(m16, m16, m16, m16, a2, d, w0, a1, t1)
    return a_out, t_wide.T


# speedup vs baseline: 4.9200x; 1.1512x over previous
import jax
import jax.numpy as jnp
from jax.experimental import pallas as pl
from jax.experimental.pallas import tpu as pltpu

NW, NT, A, T, N_STEPS = 4096, 8192, 8, 2, 5
BLK1 = 512
NB1 = NW // BLK1
BLK2 = 1024
NB2 = NW // BLK2


def _hi_lo_cols(x):
    hi = x.astype(jnp.bfloat16)
    lo = (x - hi.astype(jnp.float32)).astype(jnp.bfloat16)
    return jnp.concatenate([hi, lo], axis=1)


def _u_chain(m, s0, a_blk, rhs, a2, d):
    uu = jax.lax.dot_general(m, rhs, (((1,), (0,)), ((), ())),
                             preferred_element_type=jnp.float32)
    u = uu[:, :2] + uu[:, 2:]
    a_blk = a_blk + (s0 - u[:, 0:1]) * a2[0:1, :] + u[:, 1:2] * a2[1:2, :]
    g0 = jnp.sum(a_blk * d[:, 0][None, :], axis=1, keepdims=True)
    g1 = jnp.sum(a_blk * d[:, 1][None, :], axis=1, keepdims=True)
    return a_blk, jnp.concatenate([g0, g1], axis=1)


def _init_state(a_src, t_src, a_dst, t_s, rhs_s, s0_s):
    a_dst[...] = a_src[...]
    t0 = t_src[...]
    t_s[...] = t0
    hi = t0.astype(jnp.bfloat16)
    lo = (t0 - hi.astype(jnp.float32)).astype(jnp.bfloat16)
    rhs_s[...] = jnp.concatenate([hi, lo], axis=0).T
    s0_s[0] = jnp.sum(t0[0:1, :])


def _quarters(m_vals, base, q, a_state, rhs, a2, d, s0, v_s, cs_s):
    cs = jnp.zeros((1, A), jnp.float32)
    vv = None
    pend = None
    for k, m in enumerate(m_vals):
        aq = a_state[pl.ds(base + k * q, q), :]
        aq, gq = _u_chain(m, s0, aq, rhs, a2, d)
        a_state[pl.ds(base + k * q, q), :] = aq
        cs = cs + jnp.sum(aq, axis=0, keepdims=True)
        if pend is not None:
            vq = jax.lax.dot_general(pend[0], pend[1],
                                     (((0,), (0,)), ((), ())),
                                     preferred_element_type=jnp.float32)
            vv = vq if vv is None else vv + vq
        pend = (_hi_lo_cols(gq), m)
    vq = jax.lax.dot_general(pend[0], pend[1], (((0,), (0,)), ((), ())),
                             preferred_element_type=jnp.float32)
    vv = vq if vv is None else vv + vq
    v_s[...] += vv
    cs_s[...] += cs


def _finish_t(w0_ref, cs_s, v_s, t_s, rhs_s, s0_s):
    w0 = w0_ref[...]
    cs = cs_s[...]
    c0 = jnp.sum(cs * w0[:, 0][None, :])
    c1 = jnp.sum(cs * w0[:, 1][None, :])
    v = v_s[:2, :] + v_s[2:, :]
    c = jnp.concatenate([jnp.full((1, NT), c0, jnp.float32),
                         jnp.full((1, NT), c1, jnp.float32)], axis=0)
    t_new = t_s[...] + c + v
    t_s[...] = t_new
    hi = t_new.astype(jnp.bfloat16)
    lo = (t_new - hi.astype(jnp.float32)).astype(jnp.bfloat16)
    rhs_s[...] = jnp.concatenate([hi, lo], axis=0).T
    s0_s[0] = jnp.sum(t_new[0:1, :])


def _body1(i0_ref, i1_ref, i2_ref, i3_ref, a2_ref, d_ref, w0_ref,
           a0_ref, t0_ref, m16_out, a_out, t_out,
           v_s, cs_s, t_s, rhs_s, s0_s):
    i = pl.program_id(0)
    q = BLK1 // 4

    @pl.when(i == 0)
    def _init():
        _init_state(a0_ref, t0_ref, a_out, t_s, rhs_s, s0_s)
        v_s[...] = jnp.zeros_like(v_s)
        cs_s[...] = jnp.zeros_like(cs_s)

    ms = []
    for k, r in enumerate((i0_ref, i1_ref, i2_ref, i3_ref)):
        m = r[...].astype(jnp.float8_e4m3fn)
        m16_out[pl.ds(k * q, q), :] = m
        ms.append(m)
    _quarters(ms, i * BLK1, q, a_out, rhs_s[...], a2_ref[...], d_ref[...],
              s0_s[0], v_s, cs_s)

    @pl.when(i == NB1 - 1)
    def _finish():
        _finish_t(w0_ref, cs_s, v_s, t_s, rhs_s, s0_s)
        t_out[...] = t_s[...]


def _body2(m0_ref, m1_ref, m2_ref, m3_ref, a2_ref, d_ref, w0_ref,
           a1_ref, t1_ref, a_out, t_out, a_s, v_s, cs_s, t_s, rhs_s, s0_s):
    s = pl.program_id(0)
    i = pl.program_id(1)
    q = BLK2 // 4

    @pl.when(jnp.logical_and(s == 0, i == 0))
    def _init():
        _init_state(a1_ref, t1_ref, a_s, t_s, rhs_s, s0_s)

    @pl.when(i == 0)
    def _zero():
        v_s[...] = jnp.zeros_like(v_s)
        cs_s[...] = jnp.zeros_like(cs_s)

    ms = [m0_ref[...], m1_ref[...], m2_ref[...], m3_ref[...]]
    _quarters(ms, i * BLK2, q, a_s, rhs_s[...], a2_ref[...], d_ref[...],
              s0_s[0], v_s, cs_s)

    @pl.when(i == NB2 - 1)
    def _finish_step():
        _finish_t(w0_ref, cs_s, v_s, t_s, rhs_s, s0_s)

    @pl.when(jnp.logical_and(s == N_STEPS - 2, i == NB2 - 1))
    def _finish():
        a_out[...] = a_s[...]
        t_out[...] = t_s[...]


@jax.jit
def kernel(inputs, Awij, Awij2, first_a, first_t):
    a2 = Awij2[:, 0, :]
    d = Awij[1] - Awij[0]
    w0 = Awij[0]
    f32 = jnp.float32
    q1 = BLK1 // 4

    full1 = lambda shape: pl.BlockSpec(shape, lambda i: (0, 0))
    m16, a1, t1 = pl.pallas_call(
        _body1,
        grid=(NB1,),
        in_specs=[
            pl.BlockSpec((q1, NT), lambda i: (4 * i, 0)),
            pl.BlockSpec((q1, NT), lambda i: (4 * i + 1, 0)),
            pl.BlockSpec((q1, NT), lambda i: (4 * i + 2, 0)),
            pl.BlockSpec((q1, NT), lambda i: (4 * i + 3, 0)),
            full1((T, A)), full1((A, T)), full1((A, T)),
            full1((NW, A)), full1((T, NT)),
        ],
        out_specs=[pl.BlockSpec((BLK1, NT), lambda i: (i, 0)),
                   full1((NW, A)), full1((T, NT))],
        out_shape=[jax.ShapeDtypeStruct((NW, NT), jnp.float8_e4m3fn),
                   jax.ShapeDtypeStruct((NW, A), f32),
                   jax.ShapeDtypeStruct((T, NT), f32)],
        scratch_shapes=[
            pltpu.VMEM((4, NT), f32),
            pltpu.VMEM((1, A), f32),
            pltpu.VMEM((T, NT), f32),
            pltpu.VMEM((NT, 2 * T), jnp.bfloat16),
            pltpu.SMEM((1,), f32),
        ],
        compiler_params=pltpu.CompilerParams(
            dimension_semantics=("arbitrary",),
            vmem_limit_bytes=60 * 1024 * 1024,
        ),
    )(inputs, inputs, inputs, inputs, a2, d, w0, first_a, first_t.T)

    q2 = BLK2 // 4
    full2 = lambda shape: pl.BlockSpec(shape, lambda s, i: (0, 0))
    a_out, t_wide = pl.pallas_call(
        _body2,
        grid=(N_STEPS - 1, NB2),
        in_specs=[
            pl.BlockSpec((q2, NT), lambda s, i: (4 * i, 0)),
            pl.BlockSpec((q2, NT), lambda s, i: (4 * i + 1, 0)),
            pl.BlockSpec((q2, NT), lambda s, i: (4 * i + 2, 0)),
            pl.BlockSpec((q2, NT), lambda s, i: (4 * i + 3, 0)),
            full2((T, A)), full2((A, T)), full2((A, T)),
            full2((NW, A)), full2((T, NT)),
        ],
        out_specs=[full2((NW, A)), full2((T, NT))],
        out_shape=[jax.ShapeDtypeStruct((NW, A), f32),
                   jax.ShapeDtypeStruct((T, NT), f32)],
        scratch_shapes=[
            pltpu.VMEM((NW, A), f32),
            pltpu.VMEM((4, NT), f32),
            pltpu.VMEM((1, A), f32),
            pltpu.VMEM((T, NT), f32),
            pltpu.VMEM((NT, 2 * T), jnp.bfloat16),
            pltpu.SMEM((1,), f32),
        ],
        compiler_params=pltpu.CompilerParams(
            dimension_semantics=("arbitrary", "arbitrary"),
            vmem_limit_bytes=60 * 1024 * 1024,
        ),
    )(m16, m16, m16, m16, a2, d, w0, a1, t1)
    return a_out, t_wide.T
